# Initial kernel scaffold; baseline (speedup 1.0000x reference)
import jax
import jax.numpy as jnp
from jax.experimental import pallas as pl


def kernel(v, params):
    raise NotImplementedError("write your pallas kernel here")


# speedup vs baseline: 75.3743x; 75.3743x over previous
import jax
import jax.numpy as jnp
from jax.experimental import pallas as pl
from jax.experimental.pallas import tpu as pltpu

D = 128
HEAD = 8
HEAD_DIM = 16
KATT = 7
KCONV = 3
KK = KCONV * KCONV
B = 16
L = 1024


def _rshift(a, d):
    if d == 0:
        return a
    if d < 0:
        head = [a[:, -d - l:-d - l + 1] for l in range(-d)]
        return jnp.concatenate(head + [a[:, :L + d]], axis=1)
    tail = [a[:, 2 * (L - 1) - (l + d):2 * (L - 1) - (l + d) + 1]
            for l in range(L - d, L)]
    return jnp.concatenate([a[:, d:]] + tail, axis=1)


def _zshift(a, d):
    z = jnp.zeros((a.shape[0], abs(d)), a.dtype)
    if d < 0:
        return jnp.concatenate([z, a[:, :L + d]], axis=1)
    return jnp.concatenate([a[:, d:], z], axis=1)


def _acmix_core(xn, w1, w2, w3, wf, wd, pediff, aux, y_ref, st_ref):
    rate1 = aux[:, 2:3]
    bd = aux[:, 3:4]
    b1 = aux[:, 4:5]
    b2 = aux[:, 5:6]
    b3 = aux[:, 6:7]
    f32 = jnp.float32
    q = jnp.dot(w1, xn, preferred_element_type=f32) + b1
    k = jnp.dot(w2, xn, preferred_element_type=f32) + b2
    v = jnp.dot(w3, xn, preferred_element_type=f32) + b3

    hh = jax.lax.broadcasted_iota(jnp.int32, (HEAD, D), 0)
    hc = jax.lax.broadcasted_iota(jnp.int32, (HEAD, D), 1)
    hsum = (hc // HEAD_DIM == hh).astype(f32)
    gh = jax.lax.broadcasted_iota(jnp.int32, (D, HEAD), 1)
    gc = jax.lax.broadcasted_iota(jnp.int32, (D, HEAD), 0)
    hrep = (gc // HEAD_DIM == gh).astype(f32)

    qs = q * (float(HEAD_DIM) ** -0.5)
    atts = []
    for t in range(KATT):
        d = t - 3
        terms = qs * (_rshift(k, d) + pediff[t])
        atts.append(jnp.dot(hsum, terms, preferred_element_type=f32))
    m = atts[0]
    for a in atts[1:]:
        m = jnp.maximum(m, a)
    es = [jnp.exp(a - m) for a in atts]
    den = es[0]
    for e in es[1:]:
        den = den + e
    inv = 1.0 / den
    out_att = jnp.zeros((D, L), f32)
    for t in range(KATT):
        d = t - 3
        wfull = jnp.dot(hrep, es[t] * inv, preferred_element_type=f32)
        out_att = out_att + wfull * _rshift(v, d)

    xcat = jnp.concatenate([q, k, v], axis=0)
    fconv = jnp.dot(wf, xcat, preferred_element_type=f32)
    fbig = jnp.concatenate(
        [_zshift(fconv, -1), fconv, _zshift(fconv, 1)], axis=0)
    out_conv = jnp.dot(wd, fbig, preferred_element_type=f32) + bd

    y = jnp.maximum(rate1 * out_att + out_conv, 0.0)
    y_ref[0] = y
    ssum = jnp.sum(y, axis=1, keepdims=True)
    ssq = jnp.sum(y * y, axis=1, keepdims=True)
    st_ref[0] = jnp.concatenate(
        [ssum, ssq, jnp.zeros((D, 6), f32)], axis=1)


def _layer0_kernel(v_ref, embT_ref, w1_ref, w2_ref, w3_ref, wf_ref, wd_ref,
                   pediff_ref, aux_ref, y_ref, st_ref):
    vc = jnp.clip(v_ref[0], 0, 25)
    iota = jax.lax.broadcasted_iota(jnp.int32, (32, L), 0)
    oh = (iota == vc).astype(jnp.float32)
    xn = jnp.dot(embT_ref[...], oh, preferred_element_type=jnp.float32)
    _acmix_core(xn, w1_ref[...], w2_ref[...], w3_ref[...], wf_ref[...],
                wd_ref[...], pediff_ref[...], aux_ref[...], y_ref, st_ref)


def _layern_kernel(x_ref, w1_ref, w2_ref, w3_ref, wf_ref, wd_ref,
                   pediff_ref, aux_ref, y_ref, st_ref):
    aux = aux_ref[...]
    xn = x_ref[0] * aux[:, 0:1] + aux[:, 1:2]
    _acmix_core(xn, w1_ref[...], w2_ref[...], w3_ref[...], wf_ref[...],
                wd_ref[...], pediff_ref[...], aux=aux, y_ref=y_ref,
                st_ref=st_ref)


def _norm_kernel(x_ref, aux_ref, o_ref):
    aux = aux_ref[...]
    o_ref[0] = x_ref[0] * aux[:, 0:1] + aux[:, 1:2]


def _const_spec(shape):
    n = len(shape)
    return pl.BlockSpec(shape, lambda b: (0,) * n)


def _layer_weights(lp):
    f32 = jnp.float32
    t9 = lp['fc_w'].reshape(KK, 3, HEAD)
    wf = jnp.einsum('cph,de->cdphe', t9, jnp.eye(HEAD_DIM, dtype=f32))
    wf = wf.reshape(KK * HEAD_DIM, 3 * D)
    oh_g = (jnp.arange(HEAD_DIM)[None, :]
            == (jnp.arange(D) // (D // HEAD_DIM))[:, None]).astype(f32)
    wd = jnp.einsum('xct,xg->xtcg', lp['dep_w'], oh_g) * lp['rate2']
    wd = wd.reshape(D, KCONV * KK * HEAD_DIM)
    loc = jnp.stack([jnp.linspace(-1.0, 1.0, L),
                     -jnp.ones((L,), dtype=f32)], axis=0)
    pe = jnp.dot(lp['conv_p_w'], loc) + lp['conv_p_b'][:, None]
    pep = jnp.pad(pe, ((0, 0), (3, 3)), mode='reflect')
    pediff = jnp.stack([pe - pep[:, t:t + L] for t in range(KATT)], axis=0)
    pediff = jnp.tile(pediff, (1, HEAD, 1))
    return wf, wd, pediff


def _aux(lp, s_prev, t_prev):
    f32 = jnp.float32
    ones = jnp.ones((D,), f32)
    cols = [s_prev, t_prev, lp['rate1'] * ones, lp['rate2'] * lp['dep_b'],
            lp['conv1_b'], lp['conv2_b'], lp['conv3_b'], jnp.zeros((D,), f32)]
    return jnp.stack(cols, axis=1)


def _st_fold(st, lp):
    n = float(B * L)
    ssum = jnp.sum(st[:, :, 0], axis=0)
    ssq = jnp.sum(st[:, :, 1], axis=0)
    mean = ssum / n
    var = ssq / n - mean * mean
    s = lp['bn_g'] * jax.lax.rsqrt(var + 1e-5)
    return s, lp['bn_b'] - mean * s


def kernel(v, params):
    f32 = jnp.float32
    v3 = v.astype(jnp.int32).reshape(B, 1, L)
    embT = jnp.zeros((D, 32), f32).at[:, :26].set(params['emb'].T)

    out_shapes = [jax.ShapeDtypeStruct((B, D, L), f32),
                  jax.ShapeDtypeStruct((B, D, 8), f32)]
    out_specs = [pl.BlockSpec((1, D, L), lambda b: (b, 0, 0)),
                 pl.BlockSpec((1, D, 8), lambda b: (b, 0, 0))]
    w_specs = [_const_spec((D, D))] * 3 + [
        _const_spec((KK * HEAD_DIM, 3 * D)),
        _const_spec((D, KCONV * KK * HEAD_DIM)),
        _const_spec((KATT, D, L)),
        _const_spec((D, 8)),
    ]

    x = None
    s_prev = t_prev = None
    for i in range(3):
        lp = params['layer%d' % i]
        wf, wd, pediff = _layer_weights(lp)
        if i == 0:
            aux = _aux(lp, jnp.ones((D,), f32), jnp.zeros((D,), f32))
            x, st = pl.pallas_call(
                _layer0_kernel,
                grid=(B,),
                in_specs=[pl.BlockSpec((1, 1, L), lambda b: (b, 0, 0)),
                          _const_spec((D, 32))] + w_specs,
                out_specs=out_specs,
                out_shape=out_shapes,
            )(v3, embT, lp['conv1_w'], lp['conv2_w'], lp['conv3_w'],
              wf, wd, pediff, aux)
        else:
            aux = _aux(lp, s_prev, t_prev)
            x, st = pl.pallas_call(
                _layern_kernel,
                grid=(B,),
                in_specs=[pl.BlockSpec((1, D, L), lambda b: (b, 0, 0))]
                + w_specs,
                out_specs=out_specs,
                out_shape=out_shapes,
            )(x, lp['conv1_w'], lp['conv2_w'], lp['conv3_w'],
              wf, wd, pediff, aux)
        s_prev, t_prev = _st_fold(st, lp)

    aux = jnp.stack([s_prev, t_prev] + [jnp.zeros((D,), f32)] * 6, axis=1)
    y = pl.pallas_call(
        _norm_kernel,
        grid=(B,),
        in_specs=[pl.BlockSpec((1, D, L), lambda b: (b, 0, 0)),
                  _const_spec((D, 8))],
        out_specs=pl.BlockSpec((1, D, L), lambda b: (b, 0, 0)),
        out_shape=jax.ShapeDtypeStruct((B, D, L), f32),
    )(x, aux)
    return y.reshape(B, L, D)
